# Initial kernel scaffold; baseline (speedup 1.0000x reference)
import jax
import jax.numpy as jnp
from jax.experimental import pallas as pl


def kernel(boxes):
    raise NotImplementedError("write your pallas kernel here")


# speedup vs baseline: 3.2505x; 3.2505x over previous
import jax
import jax.numpy as jnp
from jax.experimental import pallas as pl

_THRESHOLD = 0.8
_ROW_BLOCK = 512


def _filter_kernel(rows_ref, cols_ref, fb_ref, keep_ref):
    rows = rows_ref[...]
    rx1 = rows[:, 0:1]
    ry1 = rows[:, 1:2]
    rx2 = rows[:, 2:3]
    ry2 = rows[:, 3:4]
    cx1 = cols_ref[0:1, :]
    cy1 = cols_ref[1:2, :]
    cx2 = cols_ref[2:3, :]
    cy2 = cols_ref[3:4, :]
    careas = (cx2 - cx1) * (cy2 - cy1)
    contained = (
        (cx1 >= rx1) & (cy1 >= ry1) & (cx2 <= rx2) & (cy2 <= ry2)
    )
    total = jnp.sum(jnp.where(contained, careas, 0.0), axis=1, keepdims=True)
    rarea = (rx2 - rx1) * (ry2 - ry1)
    keep = (total - rarea) <= _THRESHOLD * (rarea + 1e-9)
    keepf = keep.astype(jnp.float32)
    fb_ref[...] = rows * keepf
    keep_ref[...] = keepf


def kernel(boxes):
    n = boxes.shape[0]
    r = _ROW_BLOCK
    n_pad = ((n + r - 1) // r) * r
    bpad = jnp.pad(boxes.astype(jnp.float32), ((0, n_pad - n), (0, 0)))
    cols = bpad.T
    fb, keepf = pl.pallas_call(
        _filter_kernel,
        grid=(n_pad // r,),
        in_specs=[
            pl.BlockSpec((r, 4), lambda i: (i, 0)),
            pl.BlockSpec((4, n_pad), lambda i: (0, 0)),
        ],
        out_specs=[
            pl.BlockSpec((r, 4), lambda i: (i, 0)),
            pl.BlockSpec((r, 1), lambda i: (i, 0)),
        ],
        out_shape=[
            jax.ShapeDtypeStruct((n_pad, 4), jnp.float32),
            jax.ShapeDtypeStruct((n_pad, 1), jnp.float32),
        ],
    )(bpad, cols)
    return fb[:n], keepf[:n, 0] > 0.5
